# baseline (speedup 1.0000x reference)
import functools

import jax
import jax.numpy as jnp
from jax import lax
from jax.experimental import pallas as pl
from jax.experimental.pallas import tpu as pltpu
from jax.experimental.pallas import tpu_sc as plsc


def _sc_gather(emb, xflat, *, chunk=80):
    n_total, emb_dim = xflat.shape[0], emb.shape[1]
    info = plsc.get_sparse_core_info()
    nc, ns = info.num_cores, info.num_subcores
    nw = nc * ns
    n_per_w = n_total // nw
    assert n_per_w * nw == n_total and n_per_w % chunk == 0
    n_iters = n_per_w // chunk
    ring = 8
    ahead = 4
    assert n_iters % ring == 0 and n_iters >= 2 * ring

    mesh = plsc.VectorSubcoreMesh(core_axis_name="c", subcore_axis_name="s")

    @functools.partial(
        pl.kernel,
        mesh=mesh,
        out_type=jax.ShapeDtypeStruct((n_total, emb_dim), jnp.float32),
        scratch_types=[
            pltpu.VMEM((ring, chunk), jnp.int32),
        ] + [pltpu.VMEM((chunk, emb_dim), jnp.float32)] * ring + [
            pltpu.SemaphoreType.DMA,
            pltpu.SemaphoreType.DMA,
        ],
    )
    def gather_kernel(emb_hbm, idx_hbm, out_hbm, idx_v, *bufs_and_sems):
        rows = list(bufs_and_sems[:ring])
        sem_g, sem_o = bufs_and_sems[ring:]
        wid = lax.axis_index("s") * nc + lax.axis_index("c")
        base = wid * n_per_w

        def phase(i, p, j, n_j):
            off = base + i * chunk
            buf = rows[p]
            pltpu.make_async_copy(
                emb_hbm.at[idx_v.at[p]], buf, sem_g).wait()

            def wait_out():
                pltpu.make_async_copy(
                    rows[(p + ahead) % ring], out_hbm.at[pl.ds(off, chunk)],
                    sem_o).wait()
            if p >= ahead:
                wait_out()
            else:
                pl.when(j > 0)(wait_out)

            pltpu.async_copy(buf, out_hbm.at[pl.ds(off, chunk)], sem_o)

            @pl.when(i + ahead < n_iters)
            def _():
                nxt = (p + ahead) % ring
                pltpu.sync_copy(
                    idx_hbm.at[pl.ds(off + ahead * chunk, chunk)],
                    idx_v.at[nxt])
                pltpu.async_copy(
                    emb_hbm.at[idx_v.at[nxt]], rows[nxt], sem_g)

        for p in range(ahead):
            pltpu.sync_copy(
                idx_hbm.at[pl.ds(base + p * chunk, chunk)], idx_v.at[p])
            pltpu.async_copy(emb_hbm.at[idx_v.at[p]], rows[p], sem_g)

        n_j = n_iters // ring

        def body(j, carry):
            for p in range(ring):
                phase(ring * j + p, p, j, n_j)
            return carry

        lax.fori_loop(0, n_j, body, 0)
        for p in range(ahead):
            pltpu.make_async_copy(
                rows[(ring - ahead + p) % ring],
                out_hbm.at[pl.ds(base, chunk)], sem_o).wait()

    return gather_kernel(emb, xflat)


def _fc_in_kernel(g_ref, w_ref, b_ref, o_ref, *, c_tile):
    k = pl.program_id(0)

    @pl.when(k == 0)
    def _():
        o_ref[...] = jnp.broadcast_to(b_ref[...], o_ref.shape)

    acc = o_ref[...]
    for c in range(c_tile):
        acc += lax.dot_general(
            g_ref[:, c, :], w_ref[:, c, :], (((1,), (1,)), ((), ())),
            preferred_element_type=jnp.float32)
    o_ref[...] = acc


def _fc_in(g, w_in, b_in, *, c_tile=40):
    b, ctx, emb_dim = g.shape
    v = w_in.shape[0]
    w3 = w_in.reshape(v, ctx, emb_dim)
    n_c = ctx // c_tile
    assert n_c * c_tile == ctx
    return pl.pallas_call(
        functools.partial(_fc_in_kernel, c_tile=c_tile),
        grid=(n_c,),
        in_specs=[
            pl.BlockSpec((b, c_tile, emb_dim), lambda k: (0, k, 0)),
            pl.BlockSpec((v, c_tile, emb_dim), lambda k: (0, k, 0)),
            pl.BlockSpec((1, v), lambda k: (0, 0)),
        ],
        out_specs=pl.BlockSpec((b, v), lambda k: (0, 0)),
        out_shape=jax.ShapeDtypeStruct((b, v), jnp.float32),
    )(g, w3, b_in)


_N_Q = 4


def _fc_out_kernel(h_ref, w_hbm, b_hbm, o_hbm, s0, s1, wv0, wv1, bv0, bv1,
                   sem_w, sem_b, *sems, v_tile, n_v, rows_per):
    j = pl.program_id(0)
    wvs = [wv0, wv1]
    bvs = [bv0, bv1]

    def fetch(jj, wv, bv):
        off = pl.multiple_of(jj * v_tile, 128)
        pltpu.make_async_copy(
            w_hbm.at[pl.ds(off, v_tile), :], wv, sem_w).start()
        pltpu.make_async_copy(
            b_hbm.at[:, pl.ds(off, v_tile)], bv, sem_b).start()

    def run(scr, wv, bv, wv_nxt, bv_nxt):
        @pl.when(j == 0)
        def _():
            fetch(j, wv, bv)

        @pl.when(j + 1 < n_v)
        def _():
            fetch(j + 1, wv_nxt, bv_nxt)

        pltpu.make_async_copy(
            w_hbm.at[pl.ds(0, v_tile), :], wv, sem_w).wait()
        pltpu.make_async_copy(
            b_hbm.at[:, pl.ds(0, v_tile)], bv, sem_b).wait()

        scr[...] = lax.dot_general(
            h_ref[...], wv[...], (((1,), (1,)), ((), ())),
            preferred_element_type=jnp.float32) + bv[...]

        copies = []
        for k in range(_N_Q):
            c = pltpu.make_async_copy(
                scr.at[pl.ds(k * rows_per, rows_per), :],
                o_hbm.at[pl.ds(k * rows_per, rows_per),
                         pl.ds(pl.multiple_of(j * v_tile, 128), v_tile)],
                sems[k])
            c.start()
            copies.append(c)
        for c in copies:
            c.wait()

    @pl.when(j % 2 == 0)
    def _():
        run(s0, wv0, bv0, wv1, bv1)

    @pl.when(j % 2 == 1)
    def _():
        run(s1, wv1, bv1, wv0, bv0)


def _fc_out_tail_kernel(h_ref, w_ref, b_ref, prev_ref, o_ref):
    del prev_ref
    o_ref[...] = lax.dot_general(
        h_ref[...].astype(jnp.bfloat16),
        w_ref[...].astype(jnp.bfloat16), (((1,), (1,)), ((), ())),
        preferred_element_type=jnp.float32) + b_ref[...]


def _fc_out(h, w_out, b_out, *, v_tile=4096):
    b, v = h.shape
    vocab = w_out.shape[0]
    n_full = vocab // v_tile
    rows_per = b // _N_Q
    kern = functools.partial(
        _fc_out_kernel, v_tile=v_tile, n_v=n_full, rows_per=rows_per)
    out = pl.pallas_call(
        kern,
        grid=(n_full,),
        in_specs=[
            pl.BlockSpec((b, v), lambda j: (0, 0)),
            pl.BlockSpec(memory_space=pl.ANY),
            pl.BlockSpec(memory_space=pl.ANY),
        ],
        out_specs=pl.BlockSpec(memory_space=pl.ANY),
        out_shape=jax.ShapeDtypeStruct((b, vocab), jnp.float32),
        scratch_shapes=[
            pltpu.VMEM((b, v_tile), jnp.float32),
            pltpu.VMEM((b, v_tile), jnp.float32),
            pltpu.VMEM((v_tile, v), jnp.float32),
            pltpu.VMEM((v_tile, v), jnp.float32),
            pltpu.VMEM((1, v_tile), jnp.float32),
            pltpu.VMEM((1, v_tile), jnp.float32),
            pltpu.SemaphoreType.DMA,
            pltpu.SemaphoreType.DMA,
        ] + [pltpu.SemaphoreType.DMA] * _N_Q,
    )(h, w_out, b_out)
    if n_full * v_tile == vocab:
        return out
    return pl.pallas_call(
        _fc_out_tail_kernel,
        grid=(1,),
        in_specs=[
            pl.BlockSpec((b, v), lambda j: (0, 0)),
            pl.BlockSpec((v_tile, v), lambda j: (n_full, 0)),
            pl.BlockSpec((1, v_tile), lambda j: (0, n_full)),
            pl.BlockSpec(memory_space=pl.ANY),
        ],
        out_specs=pl.BlockSpec((b, v_tile), lambda j: (0, n_full)),
        out_shape=jax.ShapeDtypeStruct((b, vocab), jnp.float32),
        input_output_aliases={3: 0},
    )(h, w_out, b_out, out)


def kernel(x, emb, W_in, b_in, W_out, b_out):
    b, ctx = x.shape
    emb_dim = emb.shape[1]
    xflat = x.reshape(-1)
    g = _sc_gather(emb, xflat)
    g = g.reshape(b, ctx, emb_dim)
    h = _fc_in(g, W_in, b_in.reshape(1, -1))
    return _fc_out(h, W_out, b_out.reshape(1, -1))


# speedup vs baseline: 1.0994x; 1.0994x over previous
import functools

import jax
import jax.numpy as jnp
from jax import lax
from jax.experimental import pallas as pl
from jax.experimental.pallas import tpu as pltpu
from jax.experimental.pallas import tpu_sc as plsc


def _sc_gather(emb, xflat, *, chunk=80):
    n_total, emb_dim = xflat.shape[0], emb.shape[1]
    info = plsc.get_sparse_core_info()
    nc, ns = info.num_cores, info.num_subcores
    nw = nc * ns
    n_per_w = n_total // nw
    assert n_per_w * nw == n_total and n_per_w % chunk == 0
    n_iters = n_per_w // chunk
    ring = 8
    ahead = 4
    assert n_iters % ring == 0 and n_iters >= 2 * ring

    mesh = plsc.VectorSubcoreMesh(core_axis_name="c", subcore_axis_name="s")

    @functools.partial(
        pl.kernel,
        mesh=mesh,
        out_type=jax.ShapeDtypeStruct((n_total, emb_dim), jnp.float32),
        scratch_types=[
            pltpu.VMEM((ring, chunk), jnp.int32),
        ] + [pltpu.VMEM((chunk, emb_dim), jnp.float32)] * ring + [
            pltpu.SemaphoreType.DMA,
            pltpu.SemaphoreType.DMA,
        ],
    )
    def gather_kernel(emb_hbm, idx_hbm, out_hbm, idx_v, *bufs_and_sems):
        rows = list(bufs_and_sems[:ring])
        sem_g, sem_o = bufs_and_sems[ring:]
        wid = lax.axis_index("s") * nc + lax.axis_index("c")
        base = wid * n_per_w

        def phase(i, p, j, n_j):
            off = base + i * chunk
            buf = rows[p]
            pltpu.make_async_copy(
                emb_hbm.at[idx_v.at[p]], buf, sem_g).wait()

            def wait_out():
                pltpu.make_async_copy(
                    rows[(p + ahead) % ring], out_hbm.at[pl.ds(off, chunk)],
                    sem_o).wait()
            if p >= ahead:
                wait_out()
            else:
                pl.when(j > 0)(wait_out)

            pltpu.async_copy(buf, out_hbm.at[pl.ds(off, chunk)], sem_o)

            @pl.when(i + ahead < n_iters)
            def _():
                nxt = (p + ahead) % ring
                pltpu.sync_copy(
                    idx_hbm.at[pl.ds(off + ahead * chunk, chunk)],
                    idx_v.at[nxt])
                pltpu.async_copy(
                    emb_hbm.at[idx_v.at[nxt]], rows[nxt], sem_g)

        for p in range(ahead):
            pltpu.sync_copy(
                idx_hbm.at[pl.ds(base + p * chunk, chunk)], idx_v.at[p])
            pltpu.async_copy(emb_hbm.at[idx_v.at[p]], rows[p], sem_g)

        n_j = n_iters // ring

        def body(j, carry):
            for p in range(ring):
                phase(ring * j + p, p, j, n_j)
            return carry

        lax.fori_loop(0, n_j, body, 0)
        for p in range(ahead):
            pltpu.make_async_copy(
                rows[(ring - ahead + p) % ring],
                out_hbm.at[pl.ds(base, chunk)], sem_o).wait()

    return gather_kernel(emb, xflat)


def _fc_in_kernel(g_ref, w_ref, b_ref, o_ref, *, c_tile):
    k = pl.program_id(0)

    @pl.when(k == 0)
    def _():
        o_ref[...] = jnp.broadcast_to(b_ref[...], o_ref.shape)

    acc = o_ref[...]
    for c in range(c_tile):
        acc += lax.dot_general(
            g_ref[:, c, :], w_ref[:, c, :], (((1,), (1,)), ((), ())),
            preferred_element_type=jnp.float32)
    o_ref[...] = acc


def _fc_in(g, w_in, b_in, *, c_tile=40):
    b, ctx, emb_dim = g.shape
    v = w_in.shape[0]
    w3 = w_in.reshape(v, ctx, emb_dim)
    n_c = ctx // c_tile
    assert n_c * c_tile == ctx
    return pl.pallas_call(
        functools.partial(_fc_in_kernel, c_tile=c_tile),
        grid=(n_c,),
        in_specs=[
            pl.BlockSpec((b, c_tile, emb_dim), lambda k: (0, k, 0)),
            pl.BlockSpec((v, c_tile, emb_dim), lambda k: (0, k, 0)),
            pl.BlockSpec((1, v), lambda k: (0, 0)),
        ],
        out_specs=pl.BlockSpec((b, v), lambda k: (0, 0)),
        out_shape=jax.ShapeDtypeStruct((b, v), jnp.float32),
    )(g, w3, b_in)


def _fc_out_kernel(h_ref, w_ref, b_ref, o_ref):
    o_ref[...] = lax.dot_general(
        h_ref[...], w_ref[...], (((1,), (1,)), ((), ())),
        preferred_element_type=jnp.float32) + b_ref[...]


def _fc_out(h, w_out, b_out, *, v_tile=4096):
    b, v = h.shape
    vocab = w_out.shape[0]
    n_v = pl.cdiv(vocab, v_tile)
    return pl.pallas_call(
        _fc_out_kernel,
        grid=(n_v,),
        in_specs=[
            pl.BlockSpec((b, v), lambda j: (0, 0)),
            pl.BlockSpec((v_tile, v), lambda j: (j, 0)),
            pl.BlockSpec((1, v_tile), lambda j: (0, j)),
        ],
        out_specs=pl.BlockSpec((b, v_tile), lambda j: (0, j)),
        out_shape=jax.ShapeDtypeStruct((b, vocab), jnp.float32),
    )(h, w_out, b_out)


def kernel(x, emb, W_in, b_in, W_out, b_out):
    b, ctx = x.shape
    emb_dim = emb.shape[1]
    xflat = x.reshape(-1)
    g = _sc_gather(emb, xflat)
    g = g.reshape(b, ctx, emb_dim)
    h = _fc_in(g, W_in, b_in.reshape(1, -1))
    return _fc_out(h, W_out, b_out.reshape(1, -1))
